# Initial kernel scaffold; baseline (speedup 1.0000x reference)
import jax
import jax.numpy as jnp
from jax.experimental import pallas as pl


def kernel(edge_index, edge_vals, user_emb, item_emb, brand_emb):
    raise NotImplementedError("write your pallas kernel here")


# speedup vs baseline: 38.7277x; 38.7277x over previous
import functools

import jax
import jax.numpy as jnp
from jax import lax
from jax.experimental import pallas as pl
from jax.experimental.pallas import tpu as pltpu
from jax.experimental.pallas import tpu_sc as plsc

NUM_U = 50000
NUM_I = 49000
NUM_B = 1000
N = NUM_U + NUM_I + NUM_B
D = 16
E = 3200000
LAYERS = 3

NC = 2
NS = 16
NW = NC * NS

G = 80
GROUPS_PER_TILE = E // G // NW
BLKG = 25
NBLK = GROUPS_PER_TILE // BLKG
NBUF = 5
ZR = 200
NZCH = N // ZR

R128 = N * D // 128


def _spmm_body(rows_hbm, cols_hbm, vals_hbm, x_hbm, out_hbm,
               acc, rbl, cbl, vbl,
               g0b, g1b, g2b, g3b, g4b, s0b, s1b, s2b, s3b, s4b,
               zbuf, gsem, ssem):
    gath = (g0b, g1b, g2b, g3b, g4b)
    scaled = (s0b, s1b, s2b, s3b, s4b)
    c = lax.axis_index("c")
    s = lax.axis_index("s")
    zeros16 = jnp.zeros((D,), jnp.float32)

    @pl.loop(0, ZR)
    def _fill_zero(i):
        zbuf[i, :] = zeros16

    @pl.loop(0, NZCH // NS + 1)
    def _zero_acc(i):
        ch = i * NS + s

        @pl.when(ch < NZCH)
        def _():
            pltpu.sync_copy(zbuf, acc.at[pl.ds(ch * ZR, ZR)])

    plsc.subcore_barrier()

    wid = c * NS + s
    tile_b0 = wid * NBLK

    @pl.loop(0, NBLK)
    def _block(bb):
        blk = tile_b0 + bb
        pltpu.sync_copy(rows_hbm.at[blk], rbl)
        pltpu.sync_copy(cols_hbm.at[blk], cbl)
        pltpu.sync_copy(vals_hbm.at[blk], vbl)
        for b in range(NBUF):
            pltpu.async_copy(x_hbm.at[cbl.at[b]], gath[b], gsem)
            pltpu.async_copy(zbuf.at[pl.ds(0, G)], acc.at[rbl.at[b]],
                             ssem, add=True)

        @pl.loop(0, BLKG, step=NBUF)
        def _group(g0):
            for b in range(NBUF):
                g = g0 + b
                pltpu.make_async_copy(
                    x_hbm.at[pl.ds(0, G)], gath[b], gsem).wait()
                pltpu.make_async_copy(
                    x_hbm.at[pl.ds(0, G)], scaled[b], ssem).wait()
                for eg in range(G // 16):
                    vv = vbl[g, pl.ds(eg * 16, 16)]
                    for i in range(16):
                        e = eg * 16 + i
                        scaled[b][e, :] = gath[b][e, :] * vv[i]
                pltpu.async_copy(scaled[b], acc.at[rbl.at[g]],
                                 ssem, add=True)

                @pl.when(g < BLKG - NBUF)
                def _():
                    pltpu.async_copy(x_hbm.at[cbl.at[g + NBUF]],
                                     gath[b], gsem)

        for b in range(NBUF):
            pltpu.make_async_copy(
                x_hbm.at[pl.ds(0, G)], scaled[b], ssem).wait()

    plsc.subcore_barrier()

    @pl.loop(0, NZCH // NS + 1)
    def _writeback(i):
        ch = i * NS + s

        @pl.when(ch < NZCH)
        def _():
            base = ch * ZR
            pltpu.sync_copy(acc.at[pl.ds(base, ZR)], zbuf)
            pltpu.sync_copy(zbuf, out_hbm.at[c, pl.ds(base, ZR)])


_spmm = pl.kernel(
    _spmm_body,
    out_type=jax.ShapeDtypeStruct((NC, N, D), jnp.float32),
    mesh=plsc.VectorSubcoreMesh(core_axis_name="c", subcore_axis_name="s",
                                num_cores=NC, num_subcores=NS),
    compiler_params=pltpu.CompilerParams(use_tc_tiling_on_sc=False),
    scratch_types=[
        pltpu.VMEM_SHARED((N, D), jnp.float32),
        pltpu.VMEM((BLKG, G), jnp.int32),
        pltpu.VMEM((BLKG, G), jnp.int32),
        pltpu.VMEM((BLKG, G), jnp.float32),
        *[pltpu.VMEM((G, D), jnp.float32) for _ in range(NBUF)],
        *[pltpu.VMEM((G, D), jnp.float32) for _ in range(NBUF)],
        pltpu.VMEM((ZR, D), jnp.float32),
        pltpu.SemaphoreType.DMA,
        pltpu.SemaphoreType.DMA,
    ],
)


def _combine_body(p_ref, t_ref, x_ref, to_ref):
    blk = p_ref[0] + p_ref[1]
    x_ref[...] = blk
    to_ref[...] = t_ref[...] + blk


def _combine(p, tot):
    return pl.pallas_call(
        _combine_body,
        out_shape=[jax.ShapeDtypeStruct((R128, 128), jnp.float32)] * 2,
    )(p, tot)


def kernel(edge_index, edge_vals, user_emb, item_emb, brand_emb):
    ei = edge_index.astype(jnp.int32)
    rows2 = ei[0].reshape(NW * NBLK, BLKG, G)
    cols2 = ei[1].reshape(NW * NBLK, BLKG, G)
    vals2 = edge_vals.reshape(NW * NBLK, BLKG, G)
    x0 = jnp.concatenate([user_emb, item_emb, brand_emb], axis=0)
    x = x0
    tot = x0.reshape(R128, 128)
    for _ in range(LAYERS):
        part = _spmm(rows2, cols2, vals2, x)
        xf, tot = _combine(part.reshape(NC, R128, 128), tot)
        x = xf.reshape(N, D)
    final = tot.reshape(N, D) * 0.25
    return (final[:NUM_U], final[NUM_U:NUM_U + NUM_I],
            final[NUM_U + NUM_I:], user_emb, item_emb)


# speedup vs baseline: 49.2776x; 1.2724x over previous
import functools

import jax
import jax.numpy as jnp
from jax import lax
from jax.experimental import pallas as pl
from jax.experimental.pallas import tpu as pltpu
from jax.experimental.pallas import tpu_sc as plsc

NUM_U = 50000
NUM_I = 49000
NUM_B = 1000
N = NUM_U + NUM_I + NUM_B
D = 16
E = 3200000
LAYERS = 3

NC = 2
NS = 16
NW = NC * NS

G = 80
GROUPS_PER_TILE = E // G // NW
BLKG = 25
NBLK = GROUPS_PER_TILE // BLKG
NBUF = 5
ZR = 200
NZCH = N // ZR

R128 = N * D // 128


def _spmm_body(rows_hbm, cols_hbm, vals_hbm, x_hbm, out_hbm,
               acc, rbl, cbl, vbl,
               g0b, g1b, g2b, g3b, g4b, s0b, s1b, s2b, s3b, s4b,
               zbuf, gsem, ssem, isem, zsem):
    gath = (g0b, g1b, g2b, g3b, g4b)
    scaled = (s0b, s1b, s2b, s3b, s4b)
    c = lax.axis_index("c")
    s = lax.axis_index("s")
    zeros16 = jnp.zeros((D,), jnp.float32)

    @pl.loop(0, ZR)
    def _fill_zero(i):
        zbuf[i, :] = zeros16

    @pl.loop(0, NZCH // NS + 1)
    def _zero_acc(i):
        ch = i * NS + s

        @pl.when(ch < NZCH)
        def _():
            pltpu.async_copy(zbuf, acc.at[pl.ds(ch * ZR, ZR)], zsem)

    @pl.loop(0, NZCH // NS + 1)
    def _zero_drain(i):
        ch = i * NS + s

        @pl.when(ch < NZCH)
        def _():
            pltpu.make_async_copy(zbuf, acc.at[pl.ds(0, ZR)], zsem).wait()

    plsc.subcore_barrier()

    wid = c * NS + s
    tile_b0 = wid * NBLK

    pltpu.sync_copy(rows_hbm.at[tile_b0], rbl.at[0])
    pltpu.sync_copy(cols_hbm.at[tile_b0], cbl.at[0])
    pltpu.sync_copy(vals_hbm.at[tile_b0], vbl.at[0])
    for b in range(NBUF):
        pltpu.async_copy(x_hbm.at[cbl.at[0, b]], gath[b], gsem)
        pltpu.async_copy(zbuf.at[pl.ds(0, G)], acc.at[rbl.at[0, b]],
                         ssem, add=True)

    @pl.loop(0, GROUPS_PER_TILE, step=NBUF)
    def _group(g0):
        bb = g0 // BLKG
        j0 = g0 - bb * BLKG
        pb = lax.rem(bb, 2)
        not_last = bb < NBLK - 1

        @pl.when((j0 == 0) & not_last)
        def _():
            nb = tile_b0 + bb + 1
            npb = 1 - pb
            pltpu.async_copy(rows_hbm.at[nb], rbl.at[npb], isem)
            pltpu.async_copy(cols_hbm.at[nb], cbl.at[npb], isem)
            pltpu.async_copy(vals_hbm.at[nb], vbl.at[npb], isem)

        @pl.when((j0 == BLKG - NBUF) & not_last)
        def _():
            for _ in range(3):
                pltpu.make_async_copy(rows_hbm.at[tile_b0], rbl.at[0],
                                      isem).wait()

        for b in range(NBUF):
            g = g0 + b
            j = j0 + b
            pltpu.make_async_copy(
                x_hbm.at[pl.ds(0, G)], gath[b], gsem).wait()
            pltpu.make_async_copy(
                x_hbm.at[pl.ds(0, G)], scaled[b], ssem).wait()
            for eg in range(G // 16):
                vv = vbl[pb, j, pl.ds(eg * 16, 16)]
                for i in range(16):
                    e = eg * 16 + i
                    scaled[b][e, :] = gath[b][e, :] * vv[i]
            pltpu.async_copy(scaled[b], acc.at[rbl.at[pb, j]],
                             ssem, add=True)

            @pl.when(g < GROUPS_PER_TILE - NBUF)
            def _():
                h = g + NBUF
                hb = h // BLKG
                ph = lax.rem(hb, 2)
                jh = h - hb * BLKG
                pltpu.async_copy(x_hbm.at[cbl.at[ph, jh]], gath[b], gsem)

    for b in range(NBUF):
        pltpu.make_async_copy(
            x_hbm.at[pl.ds(0, G)], scaled[b], ssem).wait()

    plsc.subcore_barrier()

    @pl.loop(0, NZCH // NS + 1)
    def _writeback(i):
        ch = i * NS + s

        @pl.when(ch < NZCH)
        def _():
            base = ch * ZR
            pltpu.async_copy(acc.at[pl.ds(base, ZR)],
                             out_hbm.at[c, pl.ds(base, ZR)], zsem)

    @pl.loop(0, NZCH // NS + 1)
    def _writeback_drain(i):
        ch = i * NS + s

        @pl.when(ch < NZCH)
        def _():
            pltpu.make_async_copy(acc.at[pl.ds(0, ZR)],
                                  out_hbm.at[c, pl.ds(0, ZR)], zsem).wait()


_spmm = pl.kernel(
    _spmm_body,
    out_type=jax.ShapeDtypeStruct((NC, N, D), jnp.float32),
    mesh=plsc.VectorSubcoreMesh(core_axis_name="c", subcore_axis_name="s",
                                num_cores=NC, num_subcores=NS),
    compiler_params=pltpu.CompilerParams(use_tc_tiling_on_sc=False),
    scratch_types=[
        pltpu.VMEM_SHARED((N, D), jnp.float32),
        pltpu.VMEM((2, BLKG, G), jnp.int32),
        pltpu.VMEM((2, BLKG, G), jnp.int32),
        pltpu.VMEM((2, BLKG, G), jnp.float32),
        *[pltpu.VMEM((G, D), jnp.float32) for _ in range(NBUF)],
        *[pltpu.VMEM((G, D), jnp.float32) for _ in range(NBUF)],
        pltpu.VMEM((ZR, D), jnp.float32),
        pltpu.SemaphoreType.DMA,
        pltpu.SemaphoreType.DMA,
        pltpu.SemaphoreType.DMA,
        pltpu.SemaphoreType.DMA,
    ],
)


def _combine_body(p_ref, t_ref, x_ref, to_ref):
    blk = p_ref[0] + p_ref[1]
    x_ref[...] = blk
    to_ref[...] = t_ref[...] + blk


def _combine(p, tot):
    return pl.pallas_call(
        _combine_body,
        out_shape=[jax.ShapeDtypeStruct((R128, 128), jnp.float32)] * 2,
    )(p, tot)


def kernel(edge_index, edge_vals, user_emb, item_emb, brand_emb):
    ei = edge_index.astype(jnp.int32)
    rows2 = ei[0].reshape(NW * NBLK, BLKG, G)
    cols2 = ei[1].reshape(NW * NBLK, BLKG, G)
    vals2 = edge_vals.reshape(NW * NBLK, BLKG, G)
    x0 = jnp.concatenate([user_emb, item_emb, brand_emb], axis=0)
    x = x0
    tot = x0.reshape(R128, 128)
    for _ in range(LAYERS):
        part = _spmm(rows2, cols2, vals2, x)
        xf, tot = _combine(part.reshape(NC, R128, 128), tot)
        x = xf.reshape(N, D)
    final = tot.reshape(N, D) * 0.25
    return (final[:NUM_U], final[NUM_U:NUM_U + NUM_I],
            final[NUM_U + NUM_I:], user_emb, item_emb)


# speedup vs baseline: 49.3025x; 1.0005x over previous
import functools

import jax
import jax.numpy as jnp
from jax import lax
from jax.experimental import pallas as pl
from jax.experimental.pallas import tpu as pltpu
from jax.experimental.pallas import tpu_sc as plsc

NUM_U = 50000
NUM_I = 49000
NUM_B = 1000
N = NUM_U + NUM_I + NUM_B
D = 16
E = 3200000
LAYERS = 3

NC = 2
NS = 16
NW = NC * NS

G = 80
GROUPS_PER_TILE = E // G // NW
BLKG = 25
NBLK = GROUPS_PER_TILE // BLKG
NBUF = 5
ZR = 200
NZCH = N // ZR

R128 = N * D // 128


def _spmm_body(rows_hbm, cols_hbm, vals_hbm, x_hbm, out_hbm,
               acc, rbl, cbl, vbl,
               g0b, g1b, g2b, g3b, g4b, s0b, s1b, s2b, s3b, s4b,
               zbuf, gsem, ssem, isem, zsem):
    gath = (g0b, g1b, g2b, g3b, g4b)
    scaled = (s0b, s1b, s2b, s3b, s4b)
    c = lax.axis_index("c")
    s = lax.axis_index("s")
    zeros16 = jnp.zeros((D,), jnp.float32)

    @pl.loop(0, ZR)
    def _fill_zero(i):
        zbuf[i, :] = zeros16

    @pl.loop(0, NZCH // NS + 1)
    def _zero_acc(i):
        ch = i * NS + s

        @pl.when(ch < NZCH)
        def _():
            pltpu.async_copy(zbuf, acc.at[pl.ds(ch * ZR, ZR)], zsem)

    @pl.loop(0, NZCH // NS + 1)
    def _zero_drain(i):
        ch = i * NS + s

        @pl.when(ch < NZCH)
        def _():
            pltpu.make_async_copy(zbuf, acc.at[pl.ds(0, ZR)], zsem).wait()

    plsc.subcore_barrier()

    wid = c * NS + s
    tile_b0 = wid * NBLK

    pltpu.sync_copy(rows_hbm.at[tile_b0], rbl.at[0])
    pltpu.sync_copy(cols_hbm.at[tile_b0], cbl.at[0])
    pltpu.sync_copy(vals_hbm.at[tile_b0], vbl.at[0])
    for b in range(NBUF):
        pltpu.async_copy(x_hbm.at[cbl.at[0, b]], gath[b], gsem)
        pltpu.async_copy(zbuf.at[pl.ds(0, G)], acc.at[rbl.at[0, b]],
                         ssem, add=True)

    @pl.loop(0, GROUPS_PER_TILE, step=NBUF)
    def _group(g0):
        bb = g0 // BLKG
        j0 = g0 - bb * BLKG
        pb = lax.rem(bb, 2)
        not_last = bb < NBLK - 1

        @pl.when((j0 == 0) & not_last)
        def _():
            nb = tile_b0 + bb + 1
            npb = 1 - pb
            pltpu.async_copy(rows_hbm.at[nb], rbl.at[npb], isem)
            pltpu.async_copy(cols_hbm.at[nb], cbl.at[npb], isem)
            pltpu.async_copy(vals_hbm.at[nb], vbl.at[npb], isem)

        @pl.when((j0 == BLKG - NBUF) & not_last)
        def _():
            for _ in range(3):
                pltpu.make_async_copy(rows_hbm.at[tile_b0], rbl.at[0],
                                      isem).wait()

        for b in range(NBUF):
            g = g0 + b
            j = j0 + b
            pltpu.make_async_copy(
                x_hbm.at[pl.ds(0, G)], gath[b], gsem).wait()
            pltpu.make_async_copy(
                x_hbm.at[pl.ds(0, G)], scaled[b], ssem).wait()
            for eg in range(G // 16):
                vv = vbl[pb, j, pl.ds(eg * 16, 16)]
                for i in range(16):
                    e = eg * 16 + i
                    scaled[b][e, :] = gath[b][e, :] * vv[i]
            pltpu.async_copy(scaled[b], acc.at[rbl.at[pb, j]],
                             ssem, add=True)

            @pl.when(g < GROUPS_PER_TILE - NBUF)
            def _():
                h = g + NBUF
                hb = h // BLKG
                ph = lax.rem(hb, 2)
                jh = h - hb * BLKG
                pltpu.async_copy(x_hbm.at[cbl.at[ph, jh]], gath[b], gsem)

    for b in range(NBUF):
        pltpu.make_async_copy(
            x_hbm.at[pl.ds(0, G)], scaled[b], ssem).wait()

    plsc.subcore_barrier()

    @pl.loop(0, NZCH // NS + 1)
    def _writeback(i):
        ch = i * NS + s

        @pl.when(ch < NZCH)
        def _():
            base = ch * ZR
            pltpu.async_copy(acc.at[pl.ds(base, ZR)],
                             out_hbm.at[c, pl.ds(base, ZR)], zsem)

    @pl.loop(0, NZCH // NS + 1)
    def _writeback_drain(i):
        ch = i * NS + s

        @pl.when(ch < NZCH)
        def _():
            pltpu.make_async_copy(acc.at[pl.ds(0, ZR)],
                                  out_hbm.at[c, pl.ds(0, ZR)], zsem).wait()


_spmm = pl.kernel(
    _spmm_body,
    out_type=jax.ShapeDtypeStruct((NC, N, D), jnp.float32),
    mesh=plsc.VectorSubcoreMesh(core_axis_name="c", subcore_axis_name="s",
                                num_cores=NC, num_subcores=NS),
    compiler_params=pltpu.CompilerParams(use_tc_tiling_on_sc=False),
    scratch_types=[
        pltpu.VMEM_SHARED((N, D), jnp.float32),
        pltpu.VMEM((2, BLKG, G), jnp.int32),
        pltpu.VMEM((2, BLKG, G), jnp.int32),
        pltpu.VMEM((2, BLKG, G), jnp.float32),
        *[pltpu.VMEM((G, D), jnp.float32) for _ in range(2 * NBUF)],
        pltpu.VMEM((ZR, D), jnp.float32),
        pltpu.SemaphoreType.DMA,
        pltpu.SemaphoreType.DMA,
        pltpu.SemaphoreType.DMA,
        pltpu.SemaphoreType.DMA,
    ],
)


def _combine_body(p_ref, t_ref, x_ref, to_ref):
    blk = p_ref[0] + p_ref[1]
    x_ref[...] = blk
    to_ref[...] = t_ref[...] + blk


def _combine(p, tot):
    return pl.pallas_call(
        _combine_body,
        out_shape=[jax.ShapeDtypeStruct((R128, 128), jnp.float32)] * 2,
    )(p, tot)


def kernel(edge_index, edge_vals, user_emb, item_emb, brand_emb):
    ei = edge_index.astype(jnp.int32)
    rows2 = ei[0].reshape(NW * NBLK, BLKG, G)
    cols2 = ei[1].reshape(NW * NBLK, BLKG, G)
    vals2 = edge_vals.reshape(NW * NBLK, BLKG, G)
    x0 = jnp.concatenate([user_emb, item_emb, brand_emb], axis=0)
    x = x0
    tot = x0.reshape(R128, 128)
    for _ in range(LAYERS):
        part = _spmm(rows2, cols2, vals2, x)
        xf, tot = _combine(part.reshape(NC, R128, 128), tot)
        x = xf.reshape(N, D)
    final = tot.reshape(N, D) * 0.25
    return (final[:NUM_U], final[NUM_U:NUM_U + NUM_I],
            final[NUM_U + NUM_I:], user_emb, item_emb)
